# baseline (speedup 1.0000x reference)
import functools
import jax, jax.numpy as jnp
from jax import lax
from jax.experimental import pallas as pl
from jax.experimental.pallas import tpu as pltpu
from jax.experimental.pallas import tpu_sc as plsc


def _make_copy(T, H, dtype):
    info = plsc.get_sparse_core_info()
    NC, NS = info.num_cores, info.num_subcores
    NW = NC * NS
    rows = T // NW
    mesh = plsc.VectorSubcoreMesh(core_axis_name="c", subcore_axis_name="s")

    @functools.partial(
        pl.kernel,
        mesh=mesh,
        out_type=jax.ShapeDtypeStruct((T, H), dtype),
        scratch_types=[pltpu.SemaphoreType.DMA],
    )
    def k(table_hbm, out_hbm, sem):
        wid = lax.axis_index("s") * NC + lax.axis_index("c")
        base = wid * rows
        pltpu.make_async_copy(
            table_hbm.at[pl.ds(base, rows)], out_hbm.at[pl.ds(base, rows)], sem
        ).start()
        pltpu.make_async_copy(
            table_hbm.at[pl.ds(base, rows)], out_hbm.at[pl.ds(base, rows)], sem
        ).wait()

    return k


def kernel(x, pos_emb):
    T = x.shape[1]
    H = pos_emb.shape[1]
    out = _make_copy(T, H, pos_emb.dtype)(pos_emb[:T])
    return out[None]


# speedup vs baseline: 6.4789x; 6.4789x over previous
import functools

import jax
import jax.numpy as jnp
from jax import lax
from jax.experimental import pallas as pl
from jax.experimental.pallas import tpu as pltpu
from jax.experimental.pallas import tpu_sc as plsc


def _make_copy(T, H, dtype):
    info = plsc.get_sparse_core_info()
    NC, NS = info.num_cores, info.num_subcores
    NW = NC * NS
    rows_per_w = T // NW
    NBUF = 4
    chunk = rows_per_w // NBUF

    mesh = plsc.VectorSubcoreMesh(core_axis_name="c", subcore_axis_name="s")

    @functools.partial(
        pl.kernel,
        mesh=mesh,
        out_type=jax.ShapeDtypeStruct((T, H), dtype),
        scratch_types=[
            pltpu.VMEM((NBUF, chunk, H), dtype),
            pltpu.SemaphoreType.DMA,
            pltpu.SemaphoreType.DMA,
            pltpu.SemaphoreType.DMA,
            pltpu.SemaphoreType.DMA,
            pltpu.SemaphoreType.DMA,
        ],
    )
    def _copy(table_hbm, out_hbm, buf, s0, s1, s2, s3, out_sem):
        wid = lax.axis_index("s") * NC + lax.axis_index("c")
        base = wid * rows_per_w
        in_sems = (s0, s1, s2, s3)

        def in_copy(i):
            return pltpu.make_async_copy(
                table_hbm.at[pl.ds(base + i * chunk, chunk)],
                buf.at[i],
                in_sems[i],
            )

        def out_copy(i):
            return pltpu.make_async_copy(
                buf.at[i], out_hbm.at[pl.ds(base + i * chunk, chunk)], out_sem
            )

        for i in range(NBUF):
            in_copy(i).start()
        for i in range(NBUF):
            in_copy(i).wait()
            out_copy(i).start()
        for i in range(NBUF):
            out_copy(i).wait()

    return _copy


def kernel(x, pos_emb):
    T = x.shape[1]
    H = pos_emb.shape[1]
    out = _make_copy(T, H, pos_emb.dtype)(pos_emb[:T])
    return out[None]


# speedup vs baseline: 6.5389x; 1.0093x over previous
import functools

import jax
import jax.numpy as jnp
from jax import lax
from jax.experimental import pallas as pl
from jax.experimental.pallas import tpu as pltpu
from jax.experimental.pallas import tpu_sc as plsc


def _make_copy(T, H, dtype):
    info = plsc.get_sparse_core_info()
    NC, NS = info.num_cores, info.num_subcores
    NW = NC * NS
    rows_per_w = T // NW
    chunk = rows_per_w

    mesh = plsc.VectorSubcoreMesh(core_axis_name="c", subcore_axis_name="s")

    @functools.partial(
        pl.kernel,
        mesh=mesh,
        out_type=jax.ShapeDtypeStruct((T, H), dtype),
        scratch_types=[
            pltpu.VMEM((chunk, H), dtype),
            pltpu.SemaphoreType.DMA,
        ],
    )
    def _copy(table_hbm, out_hbm, buf, sem):
        wid = lax.axis_index("s") * NC + lax.axis_index("c")
        base = wid * rows_per_w
        pltpu.sync_copy(table_hbm.at[pl.ds(base, chunk)], buf)
        pltpu.sync_copy(buf, out_hbm.at[pl.ds(base, chunk)])

    return _copy


def kernel(x, pos_emb):
    T = x.shape[1]
    H = pos_emb.shape[1]
    out = _make_copy(T, H, pos_emb.dtype)(pos_emb[:T])
    return out[None]
